# baseline (speedup 1.0000x reference)
import jax
import jax.numpy as jnp
from jax.experimental import pallas as pl
from jax.experimental.pallas import tpu as pltpu

_TOPK = 16
_BK = 8192
_NL = 512
_NQS = 1
_IMAX = jnp.iinfo(jnp.int32).max


def _dot2(qs_ref, keys_ref):
    return jax.lax.dot_general(
        qs_ref[...], keys_ref[...],
        dimension_numbers=(((1,), (1,)), ((), ())),
        preferred_element_type=jnp.float32,
    )


def _stream_body(qn_ref, k2_ref, qs_ref, keys_ref, v_ref, i_ref):
    kb = pl.program_id(1)
    nq = qs_ref.shape[0]

    @pl.when(kb == 0)
    def _():
        v_ref[...] = jnp.full(v_ref.shape, jnp.inf, jnp.float32)
        i_ref[...] = jnp.zeros(i_ref.shape, jnp.int32)

    for g in range(_BK // _NL):
        dot = jax.lax.dot_general(
            qs_ref[...], keys_ref[g * _NL:(g + 1) * _NL, :],
            dimension_numbers=(((1,), (1,)), ((), ())),
            preferred_element_type=jnp.float32,
        )
        x = (qn_ref[...] + k2_ref[:, g * _NL:(g + 1) * _NL]) - dot
        xi = (kb * _BK + g * _NL) + jax.lax.broadcasted_iota(
            jnp.int32, (nq, _NL), 1)
        m1, m2, m3, m4 = v_ref[0], v_ref[1], v_ref[2], v_ref[3]
        i1, i2, i3 = i_ref[0], i_ref[1], i_ref[2]
        c1 = x < m1
        t = jnp.maximum(x, m1)
        nm1 = jnp.minimum(x, m1)
        ti = jnp.where(c1, i1, xi)
        ni1 = jnp.where(c1, xi, i1)
        c2 = t < m2
        t2 = jnp.maximum(t, m2)
        nm2 = jnp.minimum(t, m2)
        ti2 = jnp.where(c2, i2, ti)
        ni2 = jnp.where(c2, ti, i2)
        c3 = t2 < m3
        t3 = jnp.maximum(t2, m3)
        nm3 = jnp.minimum(t2, m3)
        ni3 = jnp.where(c3, ti2, i3)
        nm4 = jnp.minimum(t3, m4)
        v_ref[0], v_ref[1], v_ref[2], v_ref[3] = nm1, nm2, nm3, nm4
        i_ref[0], i_ref[1], i_ref[2] = ni1, ni2, ni3


def _merge_body(v_ref, i_ref, ovals_ref, oidx_ref, flag_ref):
    s = jnp.concatenate([v_ref[0], v_ref[1], v_ref[2]], axis=1)
    idx = jnp.concatenate([i_ref[0], i_ref[1], i_ref[2]], axis=1)
    ms, wis = [], []
    for _ in range(_TOPK):
        m = jnp.min(s, axis=1, keepdims=True)
        ism = s == m
        wi = jnp.min(jnp.where(ism, idx, _IMAX), axis=1, keepdims=True)
        ms.append(m)
        wis.append(wi)
        s = jnp.where(ism & (idx == wi), jnp.inf, s)
    ovals_ref[...] = jnp.concatenate(ms, axis=1)
    oidx_ref[...] = jnp.concatenate(wis, axis=1)
    unsound = jnp.any(v_ref[3] <= ms[-1])
    flag_ref[...] = jnp.broadcast_to(unsound.astype(jnp.int32),
                                     flag_ref.shape)


def _block_topk_body(qn_ref, k2_ref, qs_ref, keys_ref, vals_ref, idx_ref):
    kb = pl.program_id(0)
    nq = qs_ref.shape[0]
    bk = keys_ref.shape[0]
    s = (qn_ref[...] + k2_ref[...]) - _dot2(qs_ref, keys_ref)
    col = kb * bk + jax.lax.broadcasted_iota(jnp.int32, (nq, bk), 1)
    ms, wis = [], []
    for _ in range(_TOPK):
        m = jnp.min(s, axis=1, keepdims=True)
        ism = s == m
        wi = jnp.min(jnp.where(ism, col, _IMAX), axis=1, keepdims=True)
        ms.append(m)
        wis.append(wi)
        s = jnp.where(ism & (col == wi), jnp.inf, s)
    vals_ref[0, :, :] = jnp.concatenate(ms, axis=1)
    idx_ref[0, :, :] = jnp.concatenate(wis, axis=1)


def _final_merge_body(vals_ref, idx_ref, ovals_ref, oidx_ref):
    s = vals_ref[...]
    idx = idx_ref[...]
    ms, wis = [], []
    for _ in range(_TOPK):
        m = jnp.min(s, axis=1, keepdims=True)
        ism = s == m
        wi = jnp.min(jnp.where(ism, idx, _IMAX), axis=1, keepdims=True)
        ms.append(m)
        wis.append(wi)
        s = jnp.where(ism & (idx == wi), jnp.inf, s)
    ovals_ref[...] = jnp.concatenate(ms, axis=1)
    oidx_ref[...] = jnp.concatenate(wis, axis=1)


def _fast_path(qn, k2p, qs, keys_p, nkb):
    nq, d = qs.shape
    nqc = nq // _NQS
    v, i = pl.pallas_call(
        _stream_body,
        grid=(_NQS, nkb),
        in_specs=[
            pl.BlockSpec((nqc, 1), lambda qc, kb: (qc, 0)),
            pl.BlockSpec((1, _BK), lambda qc, kb: (0, kb)),
            pl.BlockSpec((nqc, d), lambda qc, kb: (qc, 0)),
            pl.BlockSpec((_BK, d), lambda qc, kb: (kb, 0)),
        ],
        out_specs=[
            pl.BlockSpec((4, nqc, _NL), lambda qc, kb: (0, qc, 0)),
            pl.BlockSpec((3, nqc, _NL), lambda qc, kb: (0, qc, 0)),
        ],
        out_shape=[
            jax.ShapeDtypeStruct((4, nq, _NL), jnp.float32),
            jax.ShapeDtypeStruct((3, nq, _NL), jnp.int32),
        ],
        compiler_params=pltpu.CompilerParams(
            dimension_semantics=("arbitrary", "arbitrary")),
    )(qn, k2p, qs, keys_p)

    fvals, fidx, flag = pl.pallas_call(
        _merge_body,
        grid=(_NQS,),
        in_specs=[
            pl.BlockSpec((4, nq // _NQS, _NL), lambda qc: (0, qc, 0)),
            pl.BlockSpec((3, nq // _NQS, _NL), lambda qc: (0, qc, 0)),
        ],
        out_specs=[
            pl.BlockSpec((nq // _NQS, _TOPK), lambda qc: (qc, 0)),
            pl.BlockSpec((nq // _NQS, _TOPK), lambda qc: (qc, 0)),
            pl.BlockSpec((1, 8, 128), lambda qc: (qc, 0, 0)),
        ],
        out_shape=[
            jax.ShapeDtypeStruct((nq, _TOPK), jnp.float32),
            jax.ShapeDtypeStruct((nq, _TOPK), jnp.int32),
            jax.ShapeDtypeStruct((_NQS, 8, 128), jnp.int32),
        ],
    )(v, i)
    return fvals, fidx, flag


def _slow_path(qn, k2p, qs, keys_p, nkb):
    nq, d = qs.shape
    bvals, bidx = pl.pallas_call(
        _block_topk_body,
        grid=(nkb,),
        in_specs=[
            pl.BlockSpec((nq, 1), lambda kb: (0, 0)),
            pl.BlockSpec((1, _BK), lambda kb: (0, kb)),
            pl.BlockSpec((nq, d), lambda kb: (0, 0)),
            pl.BlockSpec((_BK, d), lambda kb: (kb, 0)),
        ],
        out_specs=[
            pl.BlockSpec((1, nq, _TOPK), lambda kb: (kb, 0, 0)),
            pl.BlockSpec((1, nq, _TOPK), lambda kb: (kb, 0, 0)),
        ],
        out_shape=[
            jax.ShapeDtypeStruct((nkb, nq, _TOPK), jnp.float32),
            jax.ShapeDtypeStruct((nkb, nq, _TOPK), jnp.int32),
        ],
    )(qn, k2p, qs, keys_p)
    cvals = jnp.transpose(bvals, (1, 0, 2)).reshape(nq, nkb * _TOPK)
    cidx = jnp.transpose(bidx, (1, 0, 2)).reshape(nq, nkb * _TOPK)
    return pl.pallas_call(
        _final_merge_body,
        out_shape=[
            jax.ShapeDtypeStruct((nq, _TOPK), jnp.float32),
            jax.ShapeDtypeStruct((nq, _TOPK), jnp.int32),
        ],
    )(cvals, cidx)


@jax.jit
def _run(queries, keys):
    nq, d = queries.shape
    n_keys = keys.shape[0]
    nkb = (n_keys + _BK - 1) // _BK
    n_pad = nkb * _BK
    keys_p = jnp.pad(keys, ((0, n_pad - n_keys), (0, 0)))
    q2 = jnp.sum(queries * queries, axis=1, keepdims=True)
    k2 = jnp.sum(keys_p * keys_p, axis=1)[None, :]
    colpad = jnp.arange(n_pad, dtype=jnp.int32)[None, :] >= n_keys
    k2p = jnp.where(colpad, jnp.inf, k2)
    qs = 2.0 * queries

    fvals, fidx, flag = _fast_path(q2, k2p, qs, keys_p, nkb)
    return jax.lax.cond(
        jnp.max(flag[:, 0, 0]) > 0,
        lambda: _slow_path(q2, k2p, qs, keys_p, nkb),
        lambda: (fvals, fidx),
    )


def kernel(queries, keys, k):
    sq, idx = _run(queries, keys)
    vals = -jnp.sqrt(jnp.clip(sq, 1e-12, None))
    vals = vals + jnp.zeros((), vals.dtype) * k
    return vals, idx


# speedup vs baseline: 1.0413x; 1.0413x over previous
import jax
import jax.numpy as jnp
from jax.experimental import pallas as pl
from jax.experimental.pallas import tpu as pltpu

_TOPK = 16
_BK = 2048
_NL = 512
_NQS = 1
_IMAX = jnp.iinfo(jnp.int32).max


def _dot2(qs_ref, keys_ref):
    return jax.lax.dot_general(
        qs_ref[...], keys_ref[...],
        dimension_numbers=(((1,), (1,)), ((), ())),
        preferred_element_type=jnp.float32,
    )


def _stream_body(qn_ref, k2_ref, qs_ref, keys_ref, v_ref, i_ref):
    kb = pl.program_id(1)
    nq = qs_ref.shape[0]

    @pl.when(kb == 0)
    def _():
        v_ref[...] = jnp.full(v_ref.shape, jnp.inf, jnp.float32)
        i_ref[...] = jnp.zeros(i_ref.shape, jnp.int32)

    for g in range(_BK // _NL):
        dot = jax.lax.dot_general(
            qs_ref[...], keys_ref[g * _NL:(g + 1) * _NL, :],
            dimension_numbers=(((1,), (1,)), ((), ())),
            preferred_element_type=jnp.float32,
        )
        x = (qn_ref[...] + k2_ref[:, g * _NL:(g + 1) * _NL]) - dot
        xi = (kb * _BK + g * _NL) + jax.lax.broadcasted_iota(
            jnp.int32, (nq, _NL), 1)
        m1, m2, m3, m4 = v_ref[0], v_ref[1], v_ref[2], v_ref[3]
        i1, i2, i3 = i_ref[0], i_ref[1], i_ref[2]
        c1 = x < m1
        t = jnp.maximum(x, m1)
        nm1 = jnp.minimum(x, m1)
        ti = jnp.where(c1, i1, xi)
        ni1 = jnp.where(c1, xi, i1)
        c2 = t < m2
        t2 = jnp.maximum(t, m2)
        nm2 = jnp.minimum(t, m2)
        ti2 = jnp.where(c2, i2, ti)
        ni2 = jnp.where(c2, ti, i2)
        c3 = t2 < m3
        t3 = jnp.maximum(t2, m3)
        nm3 = jnp.minimum(t2, m3)
        ni3 = jnp.where(c3, ti2, i3)
        nm4 = jnp.minimum(t3, m4)
        v_ref[0], v_ref[1], v_ref[2], v_ref[3] = nm1, nm2, nm3, nm4
        i_ref[0], i_ref[1], i_ref[2] = ni1, ni2, ni3


def _merge_body(v_ref, i_ref, ovals_ref, oidx_ref, flag_ref):
    s = jnp.concatenate([v_ref[0], v_ref[1], v_ref[2]], axis=1)
    idx = jnp.concatenate([i_ref[0], i_ref[1], i_ref[2]], axis=1)
    ms, wis = [], []
    for _ in range(_TOPK):
        m = jnp.min(s, axis=1, keepdims=True)
        ism = s == m
        wi = jnp.min(jnp.where(ism, idx, _IMAX), axis=1, keepdims=True)
        ms.append(m)
        wis.append(wi)
        s = jnp.where(ism & (idx == wi), jnp.inf, s)
    ovals_ref[...] = jnp.concatenate(ms, axis=1)
    oidx_ref[...] = jnp.concatenate(wis, axis=1)
    unsound = jnp.any(v_ref[3] <= ms[-1])
    flag_ref[...] = jnp.broadcast_to(unsound.astype(jnp.int32),
                                     flag_ref.shape)


def _block_topk_body(qn_ref, k2_ref, qs_ref, keys_ref, vals_ref, idx_ref):
    kb = pl.program_id(0)
    nq = qs_ref.shape[0]
    bk = keys_ref.shape[0]
    s = (qn_ref[...] + k2_ref[...]) - _dot2(qs_ref, keys_ref)
    col = kb * bk + jax.lax.broadcasted_iota(jnp.int32, (nq, bk), 1)
    ms, wis = [], []
    for _ in range(_TOPK):
        m = jnp.min(s, axis=1, keepdims=True)
        ism = s == m
        wi = jnp.min(jnp.where(ism, col, _IMAX), axis=1, keepdims=True)
        ms.append(m)
        wis.append(wi)
        s = jnp.where(ism & (col == wi), jnp.inf, s)
    vals_ref[0, :, :] = jnp.concatenate(ms, axis=1)
    idx_ref[0, :, :] = jnp.concatenate(wis, axis=1)


def _final_merge_body(vals_ref, idx_ref, ovals_ref, oidx_ref):
    s = vals_ref[...]
    idx = idx_ref[...]
    ms, wis = [], []
    for _ in range(_TOPK):
        m = jnp.min(s, axis=1, keepdims=True)
        ism = s == m
        wi = jnp.min(jnp.where(ism, idx, _IMAX), axis=1, keepdims=True)
        ms.append(m)
        wis.append(wi)
        s = jnp.where(ism & (idx == wi), jnp.inf, s)
    ovals_ref[...] = jnp.concatenate(ms, axis=1)
    oidx_ref[...] = jnp.concatenate(wis, axis=1)


def _fast_path(qn, k2p, qs, keys_p, nkb):
    nq, d = qs.shape
    nqc = nq // _NQS
    v, i = pl.pallas_call(
        _stream_body,
        grid=(_NQS, nkb),
        in_specs=[
            pl.BlockSpec((nqc, 1), lambda qc, kb: (qc, 0)),
            pl.BlockSpec((1, _BK), lambda qc, kb: (0, kb)),
            pl.BlockSpec((nqc, d), lambda qc, kb: (qc, 0)),
            pl.BlockSpec((_BK, d), lambda qc, kb: (kb, 0)),
        ],
        out_specs=[
            pl.BlockSpec((4, nqc, _NL), lambda qc, kb: (0, qc, 0)),
            pl.BlockSpec((3, nqc, _NL), lambda qc, kb: (0, qc, 0)),
        ],
        out_shape=[
            jax.ShapeDtypeStruct((4, nq, _NL), jnp.float32),
            jax.ShapeDtypeStruct((3, nq, _NL), jnp.int32),
        ],
        compiler_params=pltpu.CompilerParams(
            dimension_semantics=("arbitrary", "arbitrary")),
    )(qn, k2p, qs, keys_p)

    fvals, fidx, flag = pl.pallas_call(
        _merge_body,
        grid=(_NQS,),
        in_specs=[
            pl.BlockSpec((4, nq // _NQS, _NL), lambda qc: (0, qc, 0)),
            pl.BlockSpec((3, nq // _NQS, _NL), lambda qc: (0, qc, 0)),
        ],
        out_specs=[
            pl.BlockSpec((nq // _NQS, _TOPK), lambda qc: (qc, 0)),
            pl.BlockSpec((nq // _NQS, _TOPK), lambda qc: (qc, 0)),
            pl.BlockSpec((1, 8, 128), lambda qc: (qc, 0, 0)),
        ],
        out_shape=[
            jax.ShapeDtypeStruct((nq, _TOPK), jnp.float32),
            jax.ShapeDtypeStruct((nq, _TOPK), jnp.int32),
            jax.ShapeDtypeStruct((_NQS, 8, 128), jnp.int32),
        ],
    )(v, i)
    return fvals, fidx, flag


def _slow_path(qn, k2p, qs, keys_p, nkb):
    nq, d = qs.shape
    bvals, bidx = pl.pallas_call(
        _block_topk_body,
        grid=(nkb,),
        in_specs=[
            pl.BlockSpec((nq, 1), lambda kb: (0, 0)),
            pl.BlockSpec((1, _BK), lambda kb: (0, kb)),
            pl.BlockSpec((nq, d), lambda kb: (0, 0)),
            pl.BlockSpec((_BK, d), lambda kb: (kb, 0)),
        ],
        out_specs=[
            pl.BlockSpec((1, nq, _TOPK), lambda kb: (kb, 0, 0)),
            pl.BlockSpec((1, nq, _TOPK), lambda kb: (kb, 0, 0)),
        ],
        out_shape=[
            jax.ShapeDtypeStruct((nkb, nq, _TOPK), jnp.float32),
            jax.ShapeDtypeStruct((nkb, nq, _TOPK), jnp.int32),
        ],
    )(qn, k2p, qs, keys_p)
    cvals = jnp.transpose(bvals, (1, 0, 2)).reshape(nq, nkb * _TOPK)
    cidx = jnp.transpose(bidx, (1, 0, 2)).reshape(nq, nkb * _TOPK)
    return pl.pallas_call(
        _final_merge_body,
        out_shape=[
            jax.ShapeDtypeStruct((nq, _TOPK), jnp.float32),
            jax.ShapeDtypeStruct((nq, _TOPK), jnp.int32),
        ],
    )(cvals, cidx)


@jax.jit
def _run(queries, keys):
    nq, d = queries.shape
    n_keys = keys.shape[0]
    nkb = (n_keys + _BK - 1) // _BK
    n_pad = nkb * _BK
    keys_p = jnp.pad(keys, ((0, n_pad - n_keys), (0, 0)))
    q2 = jnp.sum(queries * queries, axis=1, keepdims=True)
    k2 = jnp.sum(keys_p * keys_p, axis=1)[None, :]
    colpad = jnp.arange(n_pad, dtype=jnp.int32)[None, :] >= n_keys
    k2p = jnp.where(colpad, jnp.inf, k2)
    qs = 2.0 * queries

    fvals, fidx, flag = _fast_path(q2, k2p, qs, keys_p, nkb)
    return jax.lax.cond(
        jnp.max(flag[:, 0, 0]) > 0,
        lambda: _slow_path(q2, k2p, qs, keys_p, nkb),
        lambda: (fvals, fidx),
    )


def kernel(queries, keys, k):
    sq, idx = _run(queries, keys)
    vals = -jnp.sqrt(jnp.clip(sq, 1e-12, None))
    vals = vals + jnp.zeros((), vals.dtype) * k
    return vals, idx
